# baseline (speedup 1.0000x reference)
import jax
import jax.numpy as jnp
from jax.experimental import pallas as pl
from jax.experimental.pallas import tpu as pltpu

_MEM = 16


def _shift_mm_kernel(shifts):
    def kfn(x_ref, w_ref, b_ref, o_ref):
        x = x_ref[...]
        parts = []
        for s in shifts:
            if s == 0:
                parts.append(x)
            else:
                parts.append(jnp.concatenate([x[s:], x[:s]], axis=0))
        xc = jnp.concatenate(parts, axis=1) if len(parts) > 1 else parts[0]
        y = jax.lax.dot_general(
            xc, w_ref[...], (((1,), (0,)), ((), ())),
            preferred_element_type=jnp.float32)
        o_ref[...] = jnp.maximum(y + b_ref[...], 0.0)
    return kfn


def _shift_conv(x, w, b, shifts, rows_per_img, n_img_tile, n_out):
    total = x.shape[0]
    r = n_img_tile * rows_per_img
    grid = total // r
    return pl.pallas_call(
        _shift_mm_kernel(shifts),
        grid=(grid,),
        in_specs=[
            pl.BlockSpec((r, x.shape[1]), lambda i: (i, 0)),
            pl.BlockSpec(w.shape, lambda i: (0, 0)),
            pl.BlockSpec(b.shape, lambda i: (0, 0)),
        ],
        out_specs=pl.BlockSpec((r, n_out), lambda i: (i, 0)),
        out_shape=jax.ShapeDtypeStruct((total, n_out), jnp.float32),
        compiler_params=pltpu.CompilerParams(
            dimension_semantics=("parallel",)),
    )(x, w, b)


def _dense_kernel(x_ref, w_ref, b_ref, o_ref):
    y = jax.lax.dot_general(
        x_ref[...], w_ref[...], (((1,), (0,)), ((), ())),
        preferred_element_type=jnp.float32)
    o_ref[...] = jnp.maximum(y + b_ref[...], 0.0)


def _gru_kernel(x_ref, d_ref, h0_ref, s0_ref, wk_ref, rk_ref, bias_ref,
                seq_ref, hf_ref, sf_ref, h_scr, s_scr):
    t = pl.program_id(1)

    @pl.when(t == 0)
    def _init():
        h_scr[...] = h0_ref[...]
        s_scr[...] = s0_ref[...]

    x = x_ref[0]
    h = h_scr[...]
    wk = wk_ref[...]
    rk = rk_ref[...]
    bias = bias_ref[...]

    xz = jnp.dot(x, wk[0:512], preferred_element_type=jnp.float32)
    xr = jnp.dot(x, wk[512:1024], preferred_element_type=jnp.float32)
    xh = jnp.dot(x, wk[1024:1536], preferred_element_type=jnp.float32)
    rz = jnp.dot(h, rk[0:64], preferred_element_type=jnp.float32)
    rr = jnp.dot(h, rk[64:128], preferred_element_type=jnp.float32)
    rh = jnp.dot(h, rk[128:192], preferred_element_type=jnp.float32)

    z = jax.nn.sigmoid(xz + bias[0:1] + rz + bias[3:4])
    r = jax.nn.sigmoid(xr + bias[1:2] + rr + bias[4:5])
    hh = jnp.tanh(xh + bias[2:3] + r * (rh + bias[5:6]))
    h_new = z * h + (1.0 - z) * hh

    seq_ref[0] = h_new

    step = s_scr[...] + 1
    d = d_ref[0]
    reset = jnp.logical_or(d == 1, step % _MEM == 0)
    h_next = jnp.where(reset, 0.0, h_new)
    s_next = jnp.where(reset, 0, step)
    h_scr[...] = h_next
    s_scr[...] = s_next
    hf_ref[...] = h_next
    sf_ref[...] = s_next


def kernel(inputs, dones, state0, step0, c1_w, c1_b, c2_w, c2_b, c3_w, c3_b,
           dense_w, dense_b, gru_k, gru_rk, gru_b):
    b, t = inputs.shape[:2]
    bt = b * t

    x1 = inputs.reshape(bt, 21, 4, 21, 4, 4).transpose(0, 1, 3, 2, 4, 5)
    x1 = x1.reshape(bt * 441, 64)
    w1 = c1_w.reshape(2, 4, 2, 4, 4, 32).transpose(0, 2, 1, 3, 4, 5)
    w1 = w1.reshape(256, 32)
    y1 = _shift_conv(x1, w1, c1_b.reshape(1, 32),
                     shifts=(0, 1, 21, 22), rows_per_img=441,
                     n_img_tile=8, n_out=32)

    x2 = y1.reshape(bt, 21, 21, 32)[:, :20, :20, :]
    x2 = x2.reshape(bt, 10, 2, 10, 2, 32).transpose(0, 1, 3, 2, 4, 5)
    x2 = x2.reshape(bt * 100, 128)
    w2 = c2_w.reshape(2, 2, 2, 2, 32, 64).transpose(0, 2, 1, 3, 4, 5)
    w2 = w2.reshape(512, 64)
    y2 = _shift_conv(x2, w2, c2_b.reshape(1, 64),
                     shifts=(0, 1, 10, 11), rows_per_img=100,
                     n_img_tile=16, n_out=64)

    x3 = y2.reshape(bt, 10, 10, 64)[:, :9, :9, :].reshape(bt * 81, 64)
    w3 = c3_w.reshape(576, 64)
    y3 = _shift_conv(x3, w3, c3_b.reshape(1, 64),
                     shifts=(0, 1, 2, 9, 10, 11, 18, 19, 20),
                     rows_per_img=81, n_img_tile=16, n_out=64)

    xd = y3.reshape(bt, 9, 9, 64)[:, :7, :7, :].reshape(bt, 3136)
    rows_d = 128
    feats = pl.pallas_call(
        _dense_kernel,
        grid=(bt // rows_d,),
        in_specs=[
            pl.BlockSpec((rows_d, 3136), lambda i: (i, 0)),
            pl.BlockSpec((3136, 512), lambda i: (0, 0)),
            pl.BlockSpec((1, 512), lambda i: (0, 0)),
        ],
        out_specs=pl.BlockSpec((rows_d, 512), lambda i: (i, 0)),
        out_shape=jax.ShapeDtypeStruct((bt, 512), jnp.float32),
        compiler_params=pltpu.CompilerParams(
            dimension_semantics=("parallel",)),
    )(xd, dense_w, dense_b.reshape(1, 512))

    units = state0.shape[1]
    feats_tm = feats.reshape(b, t, 512).transpose(1, 0, 2)
    dones_tm = dones.transpose(1, 0)[:, :, None]
    wk_all = gru_k.transpose(1, 0).reshape(3, units, 512)
    wk_all = wk_all.transpose(0, 2, 1).reshape(3 * 512, units)
    rk_all = gru_rk.transpose(1, 0).reshape(3, units, units)
    rk_all = rk_all.transpose(0, 2, 1).reshape(3 * units, units)
    bias = jnp.concatenate(
        [gru_b[0].reshape(3, units), gru_b[1].reshape(3, units)],
        axis=0)
    bh = b // 2

    seq, state_f, step_f = pl.pallas_call(
        _gru_kernel,
        grid=(2, t),
        in_specs=[
            pl.BlockSpec((1, bh, 512), lambda g, s: (s, g, 0)),
            pl.BlockSpec((1, bh, 1), lambda g, s: (s, g, 0)),
            pl.BlockSpec((bh, units), lambda g, s: (g, 0)),
            pl.BlockSpec((bh, 1), lambda g, s: (g, 0)),
            pl.BlockSpec((3 * 512, units), lambda g, s: (0, 0)),
            pl.BlockSpec((3 * units, units), lambda g, s: (0, 0)),
            pl.BlockSpec((6, units), lambda g, s: (0, 0)),
        ],
        out_specs=[
            pl.BlockSpec((1, bh, units), lambda g, s: (s, g, 0)),
            pl.BlockSpec((bh, units), lambda g, s: (g, 0)),
            pl.BlockSpec((bh, 1), lambda g, s: (g, 0)),
        ],
        out_shape=[
            jax.ShapeDtypeStruct((t, b, units), jnp.float32),
            jax.ShapeDtypeStruct((b, units), jnp.float32),
            jax.ShapeDtypeStruct((b, 1), jnp.int32),
        ],
        scratch_shapes=[
            pltpu.VMEM((bh, units), jnp.float32),
            pltpu.VMEM((bh, 1), jnp.int32),
        ],
        compiler_params=pltpu.CompilerParams(
            dimension_semantics=("parallel", "arbitrary")),
    )(feats_tm, dones_tm, state0, step0[:, None], wk_all, rk_all, bias)

    y = seq.transpose(1, 0, 2)
    out = jnp.concatenate([feats.reshape(b, t, 512), y], axis=2)
    return (out, state_f, step_f.reshape(b))
